# Initial kernel scaffold; baseline (speedup 1.0000x reference)
import jax
import jax.numpy as jnp
from jax.experimental import pallas as pl


def kernel(wyck_x, embedding_table):
    raise NotImplementedError("write your pallas kernel here")


# speedup vs baseline: 3.6905x; 3.6905x over previous
import jax
import jax.numpy as jnp
from jax import lax
from jax.experimental import pallas as pl
from jax.experimental.pallas import tpu as pltpu
from jax.experimental.pallas import tpu_sc as plsc

NUM_EMB = 991
EMB = 64
BATCH = 4096
LIST = 200

NCORES = 2
NSUB = 16
NW = NCORES * NSUB
BPW = BATCH // NW
NBLK = BPW // 16
HALF = 32


def _sc_body(idx_hbm, table_hbm, out_hbm, table_v, idx_v, out_v):
    cid = lax.axis_index("c")
    sid = lax.axis_index("s")
    wid = sid * NCORES + cid
    base = wid * BPW

    pltpu.sync_copy(table_hbm, table_v)
    pltpu.sync_copy(idx_hbm.at[pl.ds(base * LIST, BPW * LIST)], idx_v)

    inv = jnp.float32(1.0 / LIST)
    lanes = lax.iota(jnp.int32, 16)

    def per_blk(t, _):
        row0 = t * 16
        idx_base = (row0 + lanes) * LIST
        out_base = (row0 + lanes) * EMB
        for half in range(2):
            def per_slot(l, accs):
                sidx = plsc.load_gather(idx_v, [idx_base + l])
                rowaddr = sidx * EMB + (half * HALF)
                return tuple(
                    accs[d] + plsc.load_gather(table_v, [rowaddr + d])
                    for d in range(HALF)
                )

            z = jnp.zeros((16,), jnp.float32)
            accs = lax.fori_loop(0, LIST, per_slot, (z,) * HALF)
            for d in range(HALF):
                plsc.store_scatter(
                    out_v, [out_base + (half * HALF) + d], accs[d] * inv
                )
        return 0

    lax.fori_loop(0, NBLK, per_blk, 0)

    pltpu.sync_copy(out_v, out_hbm.at[pl.ds(base * EMB, BPW * EMB)])


@jax.jit
def kernel(wyck_x, embedding_table):
    idx = wyck_x[:, -1].reshape(BATCH * LIST)
    table_flat = embedding_table.reshape(NUM_EMB * EMB)
    mesh = plsc.VectorSubcoreMesh(core_axis_name="c", subcore_axis_name="s")
    f = pl.kernel(
        _sc_body,
        out_type=jax.ShapeDtypeStruct((BATCH * EMB,), jnp.float32),
        mesh=mesh,
        compiler_params=pltpu.CompilerParams(needs_layout_passes=False),
        scratch_types=[
            pltpu.VMEM((NUM_EMB * EMB,), jnp.float32),
            pltpu.VMEM((BPW * LIST,), jnp.int32),
            pltpu.VMEM((BPW * EMB,), jnp.float32),
        ],
    )
    return f(idx, table_flat).reshape(BATCH, EMB)


# speedup vs baseline: 25.1980x; 6.8279x over previous
import jax
import jax.numpy as jnp
from jax import lax
from jax.experimental import pallas as pl
from jax.experimental.pallas import tpu as pltpu
from jax.experimental.pallas import tpu_sc as plsc

NUM_EMB = 991
VOCAB = 1024
EMB = 64
BATCH = 4096
LIST = 200
NGRP = 13

NCORES = 2
NSUB = 16
NW = NCORES * NSUB
BPW = BATCH // NW
CHUNK = 32
NCHUNK = BPW // CHUNK

MM_BLK = 512


def _sc_hist_body(idx_hbm, hist_hbm, idx_v, h0, h1, sem0, sem1):
    cid = lax.axis_index("c")
    sid = lax.axis_index("s")
    wid = sid * NCORES + cid
    base = wid * BPW

    pltpu.sync_copy(idx_hbm.at[pl.ds(base * LIST, BPW * LIST)], idx_v)

    ones = jnp.ones((16,), jnp.float32)
    zeros = jnp.zeros((16,), jnp.float32)
    lanes = lax.iota(jnp.int32, 16)
    tail_mask = lanes < (LIST - (NGRP - 1) * 16)

    bufs = (h0, h1)
    sems = (sem0, sem1)

    def do_chunk(c, buf, sem):
        def zero_one(z, _):
            buf[pl.ds(z * 16, 16)] = zeros
            return 0

        lax.fori_loop(0, CHUNK * VOCAB // 16, zero_one, 0)

        def row_hist(r, _):
            row_off = (c * CHUNK + r) * LIST
            for g in range(NGRP):
                sidx = idx_v[pl.ds(row_off + g * 16, 16)]
                if g < NGRP - 1:
                    plsc.addupdate_scatter(buf, [sidx + r * VOCAB], ones)
                else:
                    plsc.addupdate_scatter(
                        buf, [sidx + r * VOCAB], ones, mask=tail_mask
                    )
            return 0

        lax.fori_loop(0, CHUNK, row_hist, 0)

        return pltpu.async_copy(
            buf,
            hist_hbm.at[pl.ds((base + c * CHUNK) * VOCAB, CHUNK * VOCAB)],
            sem,
        )

    copies = []
    for c in range(NCHUNK):
        if c >= 2:
            copies[c - 2].wait()
        copies.append(do_chunk(c, bufs[c % 2], sems[c % 2]))
    copies[-2].wait()
    copies[-1].wait()


def _mm_body(h_ref, t_ref, o_ref):
    o_ref[...] = jnp.dot(
        h_ref[...], t_ref[...], preferred_element_type=jnp.float32
    ) * jnp.float32(1.0 / LIST)


@jax.jit
def kernel(wyck_x, embedding_table):
    idx = wyck_x[:, -1].reshape(BATCH * LIST)
    tpad = jnp.zeros((VOCAB, EMB), jnp.float32).at[:NUM_EMB].set(
        embedding_table
    )

    mesh = plsc.VectorSubcoreMesh(core_axis_name="c", subcore_axis_name="s")
    hist = pl.kernel(
        _sc_hist_body,
        out_type=jax.ShapeDtypeStruct((BATCH * VOCAB,), jnp.float32),
        mesh=mesh,
        compiler_params=pltpu.CompilerParams(needs_layout_passes=False),
        scratch_types=[
            pltpu.VMEM((BPW * LIST,), jnp.int32),
            pltpu.VMEM((CHUNK * VOCAB,), jnp.float32),
            pltpu.VMEM((CHUNK * VOCAB,), jnp.float32),
            pltpu.SemaphoreType.DMA,
            pltpu.SemaphoreType.DMA,
        ],
    )(idx)

    h = hist.reshape(BATCH, VOCAB)
    out = pl.pallas_call(
        _mm_body,
        out_shape=jax.ShapeDtypeStruct((BATCH, EMB), jnp.float32),
        grid=(BATCH // MM_BLK,),
        in_specs=[
            pl.BlockSpec((MM_BLK, VOCAB), lambda i: (i, 0)),
            pl.BlockSpec((VOCAB, EMB), lambda i: (0, 0)),
        ],
        out_specs=pl.BlockSpec((MM_BLK, EMB), lambda i: (i, 0)),
    )(h, tpad)
    return out


# speedup vs baseline: 42.8502x; 1.7005x over previous
import jax
import jax.numpy as jnp
from jax import lax
from jax.experimental import pallas as pl
from jax.experimental.pallas import tpu as pltpu
from jax.experimental.pallas import tpu_sc as plsc

NUM_EMB = 991
VOCAB = 1024
EMB = 64
BATCH = 4096
LIST = 200
NGRP = 13

NCORES = 2
NSUB = 16
NW = NCORES * NSUB
BPW = BATCH // NW
CHUNK = 32
NCHUNK = BPW // CHUNK

MM_BLK = 512


def _sc_hist_body(idx_hbm, hist_hbm, idx_v, h0, h1, sem0, sem1):
    cid = lax.axis_index("c")
    sid = lax.axis_index("s")
    wid = sid * NCORES + cid
    base = wid * BPW

    pltpu.sync_copy(idx_hbm.at[pl.ds(base * LIST, BPW * LIST)], idx_v)

    ones = jnp.ones((16,), jnp.float32)
    zeros = jnp.zeros((16,), jnp.float32)
    lanes = lax.iota(jnp.int32, 16)
    tail_mask = lanes < (LIST - (NGRP - 1) * 16)

    bufs = (h0, h1)
    sems = (sem0, sem1)

    def do_chunk(c, buf, sem):
        def zero_one(z, _):
            for u in range(16):
                buf[pl.ds(z * 256 + u * 16, 16)] = zeros
            return 0

        lax.fori_loop(0, CHUNK * VOCAB // 256, zero_one, 0)

        def row_hist(r, _):
            row_off = (c * CHUNK + r) * LIST
            hist_off = r * VOCAB
            for g in range(NGRP):
                sidx = idx_v[pl.ds(row_off + g * 16, 16)]
                if g < NGRP - 1:
                    plsc.addupdate_scatter(buf, [sidx + hist_off], ones)
                else:
                    plsc.addupdate_scatter(
                        buf, [sidx + hist_off], ones, mask=tail_mask
                    )
            return 0

        lax.fori_loop(0, CHUNK, row_hist, 0)

        return pltpu.async_copy(
            buf,
            hist_hbm.at[pl.ds((base + c * CHUNK) * VOCAB, CHUNK * VOCAB)],
            sem,
        )

    copies = []
    for c in range(NCHUNK):
        if c >= 2:
            copies[c - 2].wait()
        copies.append(do_chunk(c, bufs[c % 2], sems[c % 2]))
    copies[-2].wait()
    copies[-1].wait()


def _mm_body(h_ref, t_ref, o_ref):
    o_ref[...] = jnp.dot(
        h_ref[...].reshape(MM_BLK, VOCAB),
        t_ref[...],
        preferred_element_type=jnp.float32,
    ) * jnp.float32(1.0 / LIST)


@jax.jit
def kernel(wyck_x, embedding_table):
    idx = wyck_x[:, -1].reshape(BATCH * LIST)
    tpad = jnp.zeros((VOCAB, EMB), jnp.float32).at[:NUM_EMB].set(
        embedding_table
    )

    mesh = plsc.VectorSubcoreMesh(core_axis_name="c", subcore_axis_name="s")
    hist = pl.kernel(
        _sc_hist_body,
        out_type=jax.ShapeDtypeStruct((BATCH * VOCAB,), jnp.float32),
        mesh=mesh,
        compiler_params=pltpu.CompilerParams(needs_layout_passes=False),
        scratch_types=[
            pltpu.VMEM((BPW * LIST,), jnp.int32),
            pltpu.VMEM((CHUNK * VOCAB,), jnp.float32),
            pltpu.VMEM((CHUNK * VOCAB,), jnp.float32),
            pltpu.SemaphoreType.DMA,
            pltpu.SemaphoreType.DMA,
        ],
    )(idx)

    out = pl.pallas_call(
        _mm_body,
        out_shape=jax.ShapeDtypeStruct((BATCH, EMB), jnp.float32),
        grid=(BATCH // MM_BLK,),
        in_specs=[
            pl.BlockSpec((MM_BLK * VOCAB,), lambda i: (i,)),
            pl.BlockSpec((VOCAB, EMB), lambda i: (0, 0)),
        ],
        out_specs=pl.BlockSpec((MM_BLK, EMB), lambda i: (i, 0)),
    )(hist, tpad)
    return out


# speedup vs baseline: 50.0503x; 1.1680x over previous
import jax
import jax.numpy as jnp
from jax import lax
from jax.experimental import pallas as pl
from jax.experimental.pallas import tpu as pltpu
from jax.experimental.pallas import tpu_sc as plsc

NUM_EMB = 991
VOCAB = 1024
EMB = 64
BATCH = 4096
LIST = 200
NGRP = 13

NCORES = 2
NSUB = 16
NW = NCORES * NSUB
BPW = BATCH // NW
CHUNK = 32
NCHUNK = BPW // CHUNK

MM_BLK = 512


def _sc_hist_body(idx_hbm, hist_hbm, idx_v, h0, h1, sem0, sem1):
    cid = lax.axis_index("c")
    sid = lax.axis_index("s")
    wid = sid * NCORES + cid
    base = wid * BPW

    pltpu.sync_copy(idx_hbm.at[pl.ds(base, BPW)], idx_v)

    ones = jnp.ones((16,), jnp.float32)
    zeros = jnp.zeros((16,), jnp.float32)
    lanes = lax.iota(jnp.int32, 16)
    tail_mask = lanes >= 8

    bufs = (h0, h1)
    sems = (sem0, sem1)

    def do_chunk(c, buf, sem):
        def zero_one(z, _):
            for u in range(16):
                buf[pl.ds(z * 256 + u * 16, 16)] = zeros
            return 0

        lax.fori_loop(0, CHUNK * VOCAB // 256, zero_one, 0)

        def row_hist(r, _):
            row = c * CHUNK + r
            hist_off = r * VOCAB
            for g in range(NGRP):
                if g < NGRP - 1:
                    sidx = idx_v[row, pl.ds(g * 16, 16)]
                    plsc.addupdate_scatter(buf, [sidx + hist_off], ones)
                else:
                    sidx = idx_v[row, pl.ds(LIST - 16, 16)]
                    plsc.addupdate_scatter(
                        buf, [sidx + hist_off], ones, mask=tail_mask
                    )
            return 0

        lax.fori_loop(0, CHUNK, row_hist, 0)

        return pltpu.async_copy(
            buf,
            hist_hbm.at[pl.ds((base + c * CHUNK) * VOCAB, CHUNK * VOCAB)],
            sem,
        )

    copies = []
    for c in range(NCHUNK):
        if c >= 2:
            copies[c - 2].wait()
        copies.append(do_chunk(c, bufs[c % 2], sems[c % 2]))
    copies[-2].wait()
    copies[-1].wait()


def _mm_body(h_ref, t_ref, o_ref):
    o_ref[...] = jnp.dot(
        h_ref[...].reshape(MM_BLK, VOCAB),
        t_ref[...],
        preferred_element_type=jnp.float32,
    ) * jnp.float32(1.0 / LIST)


@jax.jit
def kernel(wyck_x, embedding_table):
    idx = wyck_x[:, -1]
    tpad = jnp.zeros((VOCAB, EMB), jnp.float32).at[:NUM_EMB].set(
        embedding_table
    )

    mesh = plsc.VectorSubcoreMesh(core_axis_name="c", subcore_axis_name="s")
    hist = pl.kernel(
        _sc_hist_body,
        out_type=jax.ShapeDtypeStruct((BATCH * VOCAB,), jnp.float32),
        mesh=mesh,
        compiler_params=pltpu.CompilerParams(needs_layout_passes=False),
        scratch_types=[
            pltpu.VMEM((BPW, LIST), jnp.int32),
            pltpu.VMEM((CHUNK * VOCAB,), jnp.float32),
            pltpu.VMEM((CHUNK * VOCAB,), jnp.float32),
            pltpu.SemaphoreType.DMA,
            pltpu.SemaphoreType.DMA,
        ],
    )(idx)

    out = pl.pallas_call(
        _mm_body,
        out_shape=jax.ShapeDtypeStruct((BATCH, EMB), jnp.float32),
        grid=(BATCH // MM_BLK,),
        in_specs=[
            pl.BlockSpec((MM_BLK * VOCAB,), lambda i: (i,)),
            pl.BlockSpec((VOCAB, EMB), lambda i: (0, 0)),
        ],
        out_specs=pl.BlockSpec((MM_BLK, EMB), lambda i: (i, 0)),
    )(hist, tpad)
    return out
